# baseline (speedup 1.0000x reference)
import functools

import jax
import jax.numpy as jnp
from jax import lax
from jax.experimental import pallas as pl
from jax.experimental.pallas import tpu as pltpu
from jax.experimental.pallas import tpu_sc as plsc

_SC_CORES = 2
_SC_SUBCORES = 16
_SC_WORKERS = _SC_CORES * _SC_SUBCORES
_LANES = 16
_SC_BLOCK_ROWS = 32


def _sc_partial_sums(pred2, gt2):
    rows, C = pred2.shape
    groups = C // _LANES
    mesh = plsc.VectorSubcoreMesh(core_axis_name="c", subcore_axis_name="s")

    @functools.partial(
        pl.kernel,
        mesh=mesh,
        out_type=jax.ShapeDtypeStruct((_SC_WORKERS, _LANES), jnp.float32),
        scratch_types=[pltpu.VMEM((_LANES,), jnp.float32)],
    )
    def k(pred_hbm, gt_hbm, out_hbm, acc_ref):
        wid = lax.axis_index("c") * _SC_SUBCORES + lax.axis_index("s")
        acc_ref[...] = jnp.zeros((_LANES,), jnp.float32)

        def body(pred_v, gt_v):
            @pl.loop(0, _SC_BLOCK_ROWS)
            def _(r):
                s = jnp.zeros((_LANES,), jnp.float32)
                g0 = None
                for c in range(groups):
                    sl = pl.ds(c * _LANES, _LANES)
                    g = gt_v[r, sl]
                    if c == 0:
                        g0 = g[0]
                    d = pred_v[r, sl] - g
                    s = s + d * d
                m = jnp.where(g0 != -1.0, 1.0, 0.0)
                acc_ref[...] = acc_ref[...] + s * m

        pltpu.emit_pipeline(
            body,
            grid=(rows // _SC_BLOCK_ROWS,),
            in_specs=[
                pl.BlockSpec((_SC_BLOCK_ROWS, C), lambda i: (i, 0)),
                pl.BlockSpec((_SC_BLOCK_ROWS, C), lambda i: (i, 0)),
            ],
            out_specs=[],
            core_axis_name=("c", "s"),
            dimension_semantics=(pltpu.PARALLEL,),
        )(pred_hbm, gt_hbm)

        pltpu.sync_copy(acc_ref, out_hbm.at[wid])

    return k(pred2, gt2)


def _tc_loss_kernel(pred_ref, gt_ref, out_ref, acc_ref):
    i = pl.program_id(0)

    @pl.when(i == 0)
    def _():
        acc_ref[0] = 0.0

    g = gt_ref[...]
    d = pred_ref[...] - g
    mask = (g[:, 0:1] != -1.0).astype(jnp.float32)
    acc_ref[0] += jnp.sum(d * d * mask)

    @pl.when(i == pl.num_programs(0) - 1)
    def _():
        out_ref[0, 0] = acc_ref[0]


def _tc_partial(pred2, gt2, block_rows):
    rows, C = pred2.shape
    grid = rows // block_rows
    out = pl.pallas_call(
        _tc_loss_kernel,
        grid=(grid,),
        in_specs=[
            pl.BlockSpec((block_rows, C), lambda i: (i, 0)),
            pl.BlockSpec((block_rows, C), lambda i: (i, 0)),
        ],
        out_specs=pl.BlockSpec((1, 1), lambda i: (0, 0), memory_space=pltpu.SMEM),
        out_shape=jax.ShapeDtypeStruct((1, 1), jnp.float32),
        scratch_shapes=[pltpu.SMEM((1,), jnp.float32)],
    )(pred2, gt2)
    return out[0, 0]


_SC_ROWS = 11264
_TC_BLOCK_ROWS = 1792


def kernel(pred, gt):
    B, N, C = pred.shape
    rows = B * N
    pred2 = pred.reshape(rows, C)
    gt2 = gt.reshape(rows, C)
    sc_part = _sc_partial_sums(pred2[rows - _SC_ROWS :], gt2[rows - _SC_ROWS :])
    tc_part = _tc_partial(pred2[: rows - _SC_ROWS], gt2[: rows - _SC_ROWS], _TC_BLOCK_ROWS)
    return tc_part + jnp.sum(sc_part)


# speedup vs baseline: 2.2934x; 2.2934x over previous
import functools

import jax
import jax.numpy as jnp
from jax import lax
from jax.experimental import pallas as pl
from jax.experimental.pallas import tpu as pltpu
from jax.experimental.pallas import tpu_sc as plsc

_SC_CORES = 2
_SC_SUBCORES = 16
_SC_WORKERS = _SC_CORES * _SC_SUBCORES
_LANES = 16
_SC_BLOCK_ROWS = 32


def _sc_partial_sums(pred2, gt2, start_row, n_rows):
    rows, C = pred2.shape
    groups = C // _LANES
    off_blocks = start_row // _SC_BLOCK_ROWS
    mesh = plsc.VectorSubcoreMesh(core_axis_name="c", subcore_axis_name="s")

    @functools.partial(
        pl.kernel,
        mesh=mesh,
        out_type=jax.ShapeDtypeStruct((_SC_WORKERS, _LANES), jnp.float32),
        scratch_types=[pltpu.VMEM((_LANES,), jnp.float32)],
    )
    def k(pred_hbm, gt_hbm, out_hbm, acc_ref):
        wid = lax.axis_index("c") * _SC_SUBCORES + lax.axis_index("s")
        acc_ref[...] = jnp.zeros((_LANES,), jnp.float32)

        def body(pred_v, gt_v):
            @pl.loop(0, _SC_BLOCK_ROWS)
            def _(r):
                s = jnp.zeros((_LANES,), jnp.float32)
                g0 = None
                for c in range(groups):
                    sl = pl.ds(c * _LANES, _LANES)
                    g = gt_v[r, sl]
                    if c == 0:
                        g0 = g[0]
                    d = pred_v[r, sl] - g
                    s = s + d * d
                m = jnp.where(g0 != -1.0, 1.0, 0.0)
                acc_ref[...] = acc_ref[...] + s * m

        pltpu.emit_pipeline(
            body,
            grid=(n_rows // _SC_BLOCK_ROWS,),
            in_specs=[
                pl.BlockSpec((_SC_BLOCK_ROWS, C), lambda i: (i + off_blocks, 0)),
                pl.BlockSpec((_SC_BLOCK_ROWS, C), lambda i: (i + off_blocks, 0)),
            ],
            out_specs=[],
            core_axis_name=("c", "s"),
            dimension_semantics=(pltpu.PARALLEL,),
        )(pred_hbm, gt_hbm)

        pltpu.sync_copy(acc_ref, out_hbm.at[wid])

    return k(pred2, gt2)


def _tc_loss_kernel(pred_ref, gt_ref, out_ref, acc_ref):
    i = pl.program_id(0)

    @pl.when(i == 0)
    def _():
        acc_ref[0] = 0.0

    g = gt_ref[...]
    d = pred_ref[...] - g
    mask = (g[:, 0:1] != -1.0).astype(jnp.float32)
    acc_ref[0] += jnp.sum(d * d * mask)

    @pl.when(i == pl.num_programs(0) - 1)
    def _():
        out_ref[0, 0] = acc_ref[0]


def _tc_partial(pred2, gt2, n_rows, block_rows):
    rows, C = pred2.shape
    grid = n_rows // block_rows
    out = pl.pallas_call(
        _tc_loss_kernel,
        grid=(grid,),
        in_specs=[
            pl.BlockSpec((block_rows, C), lambda i: (i, 0)),
            pl.BlockSpec((block_rows, C), lambda i: (i, 0)),
        ],
        out_specs=pl.BlockSpec((1, 1), lambda i: (0, 0), memory_space=pltpu.SMEM),
        out_shape=jax.ShapeDtypeStruct((1, 1), jnp.float32),
        scratch_shapes=[pltpu.SMEM((1,), jnp.float32)],
    )(pred2, gt2)
    return out[0, 0]


_SC_ROWS = 12288
_TC_BLOCK_ROWS = 2048


def kernel(pred, gt):
    B, N, C = pred.shape
    rows = B * N
    pred2 = pred.reshape(rows, C)
    gt2 = gt.reshape(rows, C)
    tc_rows = rows - _SC_ROWS
    sc_part = _sc_partial_sums(pred2, gt2, tc_rows, _SC_ROWS)
    tc_part = _tc_partial(pred2, gt2, tc_rows, _TC_BLOCK_ROWS)
    return tc_part + jnp.sum(sc_part)


# speedup vs baseline: 2.3000x; 1.0029x over previous
import functools

import jax
import jax.numpy as jnp
from jax import lax
from jax.experimental import pallas as pl
from jax.experimental.pallas import tpu as pltpu
from jax.experimental.pallas import tpu_sc as plsc

_SC_CORES = 2
_SC_SUBCORES = 16
_SC_WORKERS = _SC_CORES * _SC_SUBCORES
_LANES = 16
_SC_BLOCK_ROWS = 32


def _sc_partial_sums(pred2, gt2, start_row, n_rows):
    rows, C = pred2.shape
    groups = C // _LANES
    off_blocks = start_row // _SC_BLOCK_ROWS
    mesh = plsc.VectorSubcoreMesh(core_axis_name="c", subcore_axis_name="s")

    @functools.partial(
        pl.kernel,
        mesh=mesh,
        out_type=jax.ShapeDtypeStruct((_SC_WORKERS, _LANES), jnp.float32),
        scratch_types=[pltpu.VMEM((_LANES,), jnp.float32)],
    )
    def k(pred_hbm, gt_hbm, out_hbm, acc_ref):
        wid = lax.axis_index("c") * _SC_SUBCORES + lax.axis_index("s")
        acc_ref[...] = jnp.zeros((_LANES,), jnp.float32)

        def body(pred_v, gt_v):
            @pl.loop(0, _SC_BLOCK_ROWS)
            def _(r):
                s = jnp.zeros((_LANES,), jnp.float32)
                g0 = None
                for c in range(groups):
                    sl = pl.ds(c * _LANES, _LANES)
                    g = gt_v[r, sl]
                    if c == 0:
                        g0 = g[0]
                    d = pred_v[r, sl] - g
                    s = s + d * d
                m = jnp.where(g0 != -1.0, 1.0, 0.0)
                acc_ref[...] = acc_ref[...] + s * m

        pltpu.emit_pipeline(
            body,
            grid=(n_rows // _SC_BLOCK_ROWS,),
            in_specs=[
                pl.BlockSpec((_SC_BLOCK_ROWS, C), lambda i: (i + off_blocks, 0)),
                pl.BlockSpec((_SC_BLOCK_ROWS, C), lambda i: (i + off_blocks, 0)),
            ],
            out_specs=[],
            core_axis_name=("c", "s"),
            dimension_semantics=(pltpu.PARALLEL,),
        )(pred_hbm, gt_hbm)

        pltpu.sync_copy(acc_ref, out_hbm.at[wid])

    return k(pred2, gt2)


def _tc_loss_kernel(pred_ref, gt_ref, out_ref, acc_ref):
    i = pl.program_id(0)

    @pl.when(i == 0)
    def _():
        acc_ref[0] = 0.0

    g = gt_ref[...]
    d = pred_ref[...] - g
    mask = (g[:, 0:1] != -1.0).astype(jnp.float32)
    acc_ref[0] += jnp.sum(d * d * mask)

    @pl.when(i == pl.num_programs(0) - 1)
    def _():
        out_ref[0, 0] = acc_ref[0]


def _tc_partial(pred2, gt2, n_rows, block_rows):
    rows, C = pred2.shape
    grid = n_rows // block_rows
    out = pl.pallas_call(
        _tc_loss_kernel,
        grid=(grid,),
        in_specs=[
            pl.BlockSpec((block_rows, C), lambda i: (i, 0)),
            pl.BlockSpec((block_rows, C), lambda i: (i, 0)),
        ],
        out_specs=pl.BlockSpec((1, 1), lambda i: (0, 0), memory_space=pltpu.SMEM),
        out_shape=jax.ShapeDtypeStruct((1, 1), jnp.float32),
        scratch_shapes=[pltpu.SMEM((1,), jnp.float32)],
    )(pred2, gt2)
    return out[0, 0]


_SC_ROWS = 12288
_TC_BLOCK_ROWS = 2048


def kernel(pred, gt):
    B, N, C = pred.shape
    rows = B * N
    pred2 = pred.reshape(rows, C)
    gt2 = gt.reshape(rows, C)
    tc_rows = rows - _SC_ROWS
    tc_part = _tc_partial(pred2, gt2, tc_rows, _TC_BLOCK_ROWS)
    sc_part = _sc_partial_sums(pred2, gt2, tc_rows, _SC_ROWS)
    return tc_part + jnp.sum(sc_part)
